# Initial kernel scaffold; baseline (speedup 1.0000x reference)
import jax
import jax.numpy as jnp
from jax.experimental import pallas as pl


def kernel(x, edge_index, batch, Wrel1, Wroot1, b1, Wrel, Wroot, b, Wout, bout):
    raise NotImplementedError("write your pallas kernel here")


# speedup vs baseline: 4.2253x; 4.2253x over previous
import functools

import jax
import jax.numpy as jnp
from jax import lax
from jax.experimental import pallas as pl
from jax.experimental.pallas import tpu as pltpu
from jax.experimental.pallas import tpu_sc as plsc

_N = 10000
_E = 320000
_DIN = 128
_H = 256
_NG = 64
_NOUT = 24

_NC = 2
_NS = 16
_HALF = _N // _NC
_ZSTRIPE = 320
_SH_ROWS = _NS * _ZSTRIPE
_OSTRIPE = 312
_TRASH = _HALF
_DCOL = 128

_EB = 2048
_NB = 10
_EPT_PAD = _EB * _NB
_K = 128
_CP_ROWS = 161


def _aggregate(feats, src_pad, dst_pad, zrows):
    np_ = len(feats)
    mesh = plsc.VectorSubcoreMesh(core_axis_name="c", subcore_axis_name="s")

    @functools.partial(
        pl.kernel,
        mesh=mesh,
        compiler_params=pltpu.CompilerParams(needs_layout_passes=False),
        out_type=tuple(jax.ShapeDtypeStruct((_N, _DCOL), jnp.float32)
                       for _ in range(np_)),
        scratch_types=[
            pltpu.VMEM((_EB,), jnp.int32),
            pltpu.VMEM((_EB,), jnp.int32),
            pltpu.VMEM((_CP_ROWS, _K), jnp.int32),
            pltpu.VMEM((_CP_ROWS, _K), jnp.int32),
            pltpu.VMEM((_K, _DCOL), jnp.float32),
            pltpu.VMEM((16,), jnp.int32),
            pltpu.VMEM_SHARED((_SH_ROWS, _DCOL), jnp.float32),
            pltpu.SemaphoreType.DMA,
        ],
    )
    def k(*refs):
        feat_hbm = refs[:np_]
        src_hbm, dst_hbm, zero_hbm = refs[np_:np_ + 3]
        out_hbm = refs[np_ + 3:2 * np_ + 3]
        (src_blk, dst_blk, src_cp, dst_cp, rows_v, cnt_ref, acc, sem) = \
            refs[2 * np_ + 3:]

        c = lax.axis_index("c")
        s = lax.axis_index("s")
        base = c * _HALF

        pltpu.sync_copy(zero_hbm, acc.at[pl.ds(s * _ZSTRIPE, _ZSTRIPE)])

        cnt_ref[...] = jnp.zeros((16,), jnp.int32)
        for blk in range(_NB):
            pltpu.sync_copy(src_hbm.at[s, pl.ds(blk * _EB, _EB)], src_blk)
            pltpu.sync_copy(dst_hbm.at[s, pl.ds(blk * _EB, _EB)], dst_blk)

            def cbody(i, carry):
                cnt = cnt_ref[...]
                sv = src_blk[pl.ds(i * 16, 16)]
                dv = dst_blk[pl.ds(i * 16, 16)]
                dloc = dv - jnp.full((16,), base, jnp.int32)
                m = (dloc >= 0) & (dloc < _HALF)
                pos = cnt + plsc.cumsum(m.astype(jnp.int32)) - 1
                plsc.store_scatter(src_cp, [pos // _K, pos % _K], sv, mask=m)
                plsc.store_scatter(dst_cp, [pos // _K, pos % _K], dloc, mask=m)
                cnt_ref[...] = cnt + plsc.all_reduce_population_count(m)
                return carry

            lax.fori_loop(0, _EB // 16, cbody, 0)

        cnt = cnt_ref[...]
        lanes = lax.iota(jnp.int32, 16)
        for t in range(_K // 16):
            pos = cnt + lanes + 16 * t
            plsc.store_scatter(src_cp, [pos // _K, pos % _K],
                               jnp.zeros((16,), jnp.int32))
            plsc.store_scatter(dst_cp, [pos // _K, pos % _K],
                               jnp.full((16,), _TRASH, jnp.int32))
        nch = (cnt[0] + _K - 1) // _K

        plsc.subcore_barrier()

        for p in range(np_):
            def gbody(j, carry):
                pltpu.async_copy(feat_hbm[p].at[src_cp.at[j]], rows_v, sem).wait()
                pltpu.sync_copy(rows_v, acc.at[dst_cp.at[j]], add=True)
                return carry

            lax.fori_loop(0, nch, gbody, 0)

            plsc.subcore_barrier()

            start = s * _OSTRIPE

            @pl.when(s < _NS - 1)
            def _():
                pltpu.sync_copy(acc.at[pl.ds(start, _OSTRIPE)],
                                out_hbm[p].at[pl.ds(base + start, _OSTRIPE)])

            @pl.when(s == _NS - 1)
            def _():
                last = _HALF - (_NS - 1) * _OSTRIPE
                pltpu.sync_copy(acc.at[pl.ds(start, last)],
                                out_hbm[p].at[pl.ds(base + start, last)])

            if p + 1 < np_:
                pltpu.sync_copy(zero_hbm,
                                acc.at[pl.ds(s * _ZSTRIPE, _ZSTRIPE)])
                plsc.subcore_barrier()

    return k(*feats, src_pad, dst_pad, zrows)


def _layer_tc(aggr_halves, h, wr_halves, wro, bias):
    nh = len(aggr_halves)
    dh = h.shape[1]
    blk = 1000
    grid = _N // blk

    def body(*refs):
        a_refs = refs[:nh]
        h_ref = refs[nh]
        wr_refs = refs[nh + 1:2 * nh + 1]
        wro_ref, b_ref, o_ref = refs[2 * nh + 1:]
        acc = jnp.dot(h_ref[...], wro_ref[...], preferred_element_type=jnp.float32)
        for i in range(nh):
            acc += jnp.dot(a_refs[i][...], wr_refs[i][...],
                           preferred_element_type=jnp.float32)
        o_ref[...] = jnp.maximum(acc + b_ref[...], 0.0)

    in_specs = (
        [pl.BlockSpec((blk, _DCOL), lambda i: (i, 0)) for _ in range(nh)]
        + [pl.BlockSpec((blk, dh), lambda i: (i, 0))]
        + [pl.BlockSpec((_DCOL, _H), lambda i: (0, 0)) for _ in range(nh)]
        + [pl.BlockSpec((dh, _H), lambda i: (0, 0)),
           pl.BlockSpec((1, _H), lambda i: (0, 0))]
    )
    return pl.pallas_call(
        body,
        grid=(grid,),
        in_specs=in_specs,
        out_specs=pl.BlockSpec((blk, _H), lambda i: (i, 0)),
        out_shape=jax.ShapeDtypeStruct((_N, _H), jnp.float32),
    )(*aggr_halves, h, *wr_halves, wro, bias.reshape(1, _H))


def _pool_head(h, batch_col, wout_pad, bout_pad):
    def body(h_ref, b_ref, w_ref, bo_ref, o_ref):
        onehot = (b_ref[...] == lax.broadcasted_iota(jnp.int32, (_N, _NG), 1))
        onehot = onehot.astype(jnp.float32)
        sums = lax.dot_general(onehot, h_ref[...], (((0,), (0,)), ((), ())),
                               preferred_element_type=jnp.float32)
        counts = jnp.sum(onehot, axis=0)[:, None]
        pooled = sums / jnp.maximum(counts, 1.0)
        o_ref[...] = jnp.dot(pooled, w_ref[...],
                             preferred_element_type=jnp.float32) + bo_ref[...]

    return pl.pallas_call(
        body,
        out_shape=jax.ShapeDtypeStruct((_NG, 128), jnp.float32),
    )(h, batch_col, wout_pad, bout_pad)


def kernel(x, edge_index, batch, Wrel1, Wroot1, b1, Wrel, Wroot, b, Wout, bout):
    src = edge_index[0].astype(jnp.int32)
    dst = edge_index[1].astype(jnp.int32)
    pad = _NS * _EPT_PAD - _E
    src_pad = jnp.concatenate([src, jnp.zeros((pad,), jnp.int32)])
    dst_pad = jnp.concatenate([dst, jnp.full((pad,), 2 * _N, jnp.int32)])
    src_pad = src_pad.reshape(_NS, _EPT_PAD)
    dst_pad = dst_pad.reshape(_NS, _EPT_PAD)

    zrows = jnp.zeros((_ZSTRIPE, _DCOL), jnp.float32)

    (a,) = _aggregate((x,), src_pad, dst_pad, zrows)
    h = _layer_tc((a,), x, (Wrel1,), Wroot1, b1)
    for i in range(6):
        halves = _aggregate((h[:, :_DCOL], h[:, _DCOL:]), src_pad, dst_pad, zrows)
        h = _layer_tc(halves, h, (Wrel[i, :_DCOL], Wrel[i, _DCOL:]),
                      Wroot[i], b[i])

    batch_col = batch.astype(jnp.int32).reshape(_N, 1)
    wout_pad = jnp.zeros((_H, 128), jnp.float32).at[:, :_NOUT].set(Wout)
    bout_pad = jnp.zeros((1, 128), jnp.float32).at[0, :_NOUT].set(bout)
    out = _pool_head(h, batch_col, wout_pad, bout_pad)
    return out[:, :_NOUT]
